# baseline (speedup 1.0000x reference)
import functools

import jax
import jax.numpy as jnp
from jax.experimental import pallas as pl
from jax.experimental.pallas import tpu as pltpu

E = 8
D_MODEL = 1024
D_FF = 4096
T = 32

CF = 2048
NC = D_FF // CF
NSTEP = E * NC
NBUF = 2

EPAD = 128


def _copies(w1_hbm, w3_hbm, w2_hbm, w1_buf, w3_buf, w2_buf, sems, s):
    e = s // NC
    c = s % NC
    b = jax.lax.rem(s, NBUF)
    c1 = pltpu.make_async_copy(
        w1_hbm.at[e, pl.ds(c * CF, CF), :], w1_buf.at[b], sems.at[b, 0])
    c3 = pltpu.make_async_copy(
        w3_hbm.at[e, pl.ds(c * CF, CF), :], w3_buf.at[b], sems.at[b, 1])
    c2 = pltpu.make_async_copy(
        w2_hbm.at[e, :, pl.ds(c * CF, CF)], w2_buf.at[b], sems.at[b, 2])
    return c1, c3, c2, b


def _moe_body(x_ref, wg_ref, w1_hbm, w3_hbm, w2_hbm, out_ref,
              w1_buf, w3_buf, w2_buf, sems):
    x = x_ref[...]
    wg = wg_ref[...]

    logits = jax.lax.dot_general(
        x, wg, (((1,), (1,)), ((), ())),
        preferred_element_type=jnp.float32,
        precision=jax.lax.Precision.HIGHEST)
    col = jax.lax.broadcasted_iota(jnp.int32, logits.shape, 1)
    neg = jnp.float32(-jnp.inf)
    logits = jnp.where(col < E, logits, neg)
    m1 = jnp.max(logits, axis=1, keepdims=True)
    first = jnp.min(jnp.where(logits == m1, col, EPAD), axis=1, keepdims=True)
    m2 = jnp.max(jnp.where(col == first, neg, logits), axis=1, keepdims=True)
    sel = logits >= m2
    denom = 1.0 + jnp.exp(m2 - m1)
    wmat = jnp.where(sel, jnp.exp(logits - m1) / denom, 0.0)

    for s in range(NBUF):
        c1, c3, c2, _ = _copies(w1_hbm, w3_hbm, w2_hbm,
                                w1_buf, w3_buf, w2_buf, sems, s)
        c1.start()
        c3.start()
        c2.start()

    def step(s, acc):
        c1, c3, c2, b = _copies(w1_hbm, w3_hbm, w2_hbm,
                                w1_buf, w3_buf, w2_buf, sems, s)
        c1.wait()
        c3.wait()
        c2.wait()
        w1 = w1_buf[b]
        w3 = w3_buf[b]
        a = jax.lax.dot_general(x, w1, (((1,), (1,)), ((), ())),
                                preferred_element_type=jnp.float32)
        g = jax.lax.dot_general(x, w3, (((1,), (1,)), ((), ())),
                                preferred_element_type=jnp.float32)
        h = a * jax.nn.sigmoid(a) * g
        w2 = w2_buf[b]
        p = jax.lax.dot_general(h, w2, (((1,), (1,)), ((), ())),
                                preferred_element_type=jnp.float32)
        e = s // NC
        onehot = (jax.lax.broadcasted_iota(jnp.int32, (EPAD, 1), 0) == e
                  ).astype(jnp.float32)
        wi = jax.lax.dot_general(wmat, onehot, (((1,), (0,)), ((), ())),
                                 preferred_element_type=jnp.float32)

        @pl.when(s + NBUF < NSTEP)
        def _prefetch():
            n1, n3, n2, _ = _copies(w1_hbm, w3_hbm, w2_hbm,
                                    w1_buf, w3_buf, w2_buf, sems, s + NBUF)
            n1.start()
            n3.start()
            n2.start()

        return acc + wi * p

    acc0 = jnp.zeros((T, D_MODEL), jnp.float32)
    out_ref[...] = jax.lax.fori_loop(0, NSTEP, step, acc0)


@functools.partial(jax.jit, static_argnames=())
def kernel(inputs, Wg, W1, W2, W3):
    x = inputs.reshape(-1, inputs.shape[-1]).astype(jnp.float32)
    wg_pad = jnp.zeros((EPAD, D_MODEL), jnp.float32).at[:E].set(Wg)

    out = pl.pallas_call(
        _moe_body,
        in_specs=[
            pl.BlockSpec((T, D_MODEL), lambda: (0, 0)),
            pl.BlockSpec((EPAD, D_MODEL), lambda: (0, 0)),
            pl.BlockSpec(memory_space=pl.ANY),
            pl.BlockSpec(memory_space=pl.ANY),
            pl.BlockSpec(memory_space=pl.ANY),
        ],
        out_specs=pl.BlockSpec((T, D_MODEL), lambda: (0, 0)),
        out_shape=jax.ShapeDtypeStruct((T, D_MODEL), jnp.float32),
        scratch_shapes=[
            pltpu.VMEM((NBUF, CF, D_MODEL), jnp.float32),
            pltpu.VMEM((NBUF, CF, D_MODEL), jnp.float32),
            pltpu.VMEM((NBUF, D_MODEL, CF), jnp.float32),
            pltpu.SemaphoreType.DMA((NBUF, 3)),
        ],
        compiler_params=pltpu.CompilerParams(
            vmem_limit_bytes=128 * 1024 * 1024,
        ),
    )(x, wg_pad, W1, W3, W2)
    return out.reshape(inputs.shape)


# speedup vs baseline: 1.0240x; 1.0240x over previous
import functools

import jax
import jax.numpy as jnp
from jax.experimental import pallas as pl
from jax.experimental.pallas import tpu as pltpu

E = 8
D_MODEL = 1024
D_FF = 4096
T = 32

CF = 1024
NC = D_FF // CF
NSTEP = E * NC
NBUF = 3

EPAD = 128


def _copies(w1_hbm, w3_hbm, w2_hbm, w1_buf, w3_buf, w2_buf, sems, s):
    e = s // NC
    c = s % NC
    b = jax.lax.rem(s, NBUF)
    c1 = pltpu.make_async_copy(
        w1_hbm.at[e, pl.ds(c * CF, CF), :], w1_buf.at[b], sems.at[b, 0])
    c3 = pltpu.make_async_copy(
        w3_hbm.at[e, pl.ds(c * CF, CF), :], w3_buf.at[b], sems.at[b, 1])
    c2 = pltpu.make_async_copy(
        w2_hbm.at[e, :, pl.ds(c * CF, CF)], w2_buf.at[b], sems.at[b, 2])
    return c1, c3, c2, b


def _moe_body(x_ref, wg_ref, w1_hbm, w3_hbm, w2_hbm, out_ref,
              w1_buf, w3_buf, w2_buf, sems):
    x = x_ref[...]
    wg = wg_ref[...]

    logits = jax.lax.dot_general(
        x, wg, (((1,), (1,)), ((), ())),
        preferred_element_type=jnp.float32,
        precision=jax.lax.Precision.HIGHEST)
    col = jax.lax.broadcasted_iota(jnp.int32, logits.shape, 1)
    neg = jnp.float32(-jnp.inf)
    logits = jnp.where(col < E, logits, neg)
    m1 = jnp.max(logits, axis=1, keepdims=True)
    first = jnp.min(jnp.where(logits == m1, col, EPAD), axis=1, keepdims=True)
    m2 = jnp.max(jnp.where(col == first, neg, logits), axis=1, keepdims=True)
    sel = logits >= m2
    denom = 1.0 + jnp.exp(m2 - m1)
    wmat = jnp.where(sel, jnp.exp(logits - m1) / denom, 0.0)

    for s in range(NBUF):
        c1, c3, c2, _ = _copies(w1_hbm, w3_hbm, w2_hbm,
                                w1_buf, w3_buf, w2_buf, sems, s)
        c2.start()
        c1.start()
        c3.start()

    def step(s, acc):
        c1, c3, c2, b = _copies(w1_hbm, w3_hbm, w2_hbm,
                                w1_buf, w3_buf, w2_buf, sems, s)
        c1.wait()
        c3.wait()
        c2.wait()
        w1 = w1_buf[b]
        w3 = w3_buf[b]
        a = jax.lax.dot_general(x, w1, (((1,), (1,)), ((), ())),
                                preferred_element_type=jnp.float32)
        g = jax.lax.dot_general(x, w3, (((1,), (1,)), ((), ())),
                                preferred_element_type=jnp.float32)
        h = a * jax.nn.sigmoid(a) * g
        w2 = w2_buf[b]
        p = jax.lax.dot_general(h, w2, (((1,), (1,)), ((), ())),
                                preferred_element_type=jnp.float32)
        e = s // NC
        onehot = (jax.lax.broadcasted_iota(jnp.int32, (EPAD, 1), 0) == e
                  ).astype(jnp.float32)
        wi = jax.lax.dot_general(wmat, onehot, (((1,), (0,)), ((), ())),
                                 preferred_element_type=jnp.float32)

        @pl.when(s + NBUF < NSTEP)
        def _prefetch():
            n1, n3, n2, _ = _copies(w1_hbm, w3_hbm, w2_hbm,
                                    w1_buf, w3_buf, w2_buf, sems, s + NBUF)
            n2.start()
            n1.start()
            n3.start()

        return acc + wi * p

    acc0 = jnp.zeros((T, D_MODEL), jnp.float32)
    out_ref[...] = jax.lax.fori_loop(0, NSTEP, step, acc0)


@functools.partial(jax.jit, static_argnames=())
def kernel(inputs, Wg, W1, W2, W3):
    x = inputs.reshape(-1, inputs.shape[-1]).astype(jnp.float32)
    wg_pad = jnp.zeros((EPAD, D_MODEL), jnp.float32).at[:E].set(Wg)

    out = pl.pallas_call(
        _moe_body,
        in_specs=[
            pl.BlockSpec((T, D_MODEL), lambda: (0, 0)),
            pl.BlockSpec((EPAD, D_MODEL), lambda: (0, 0)),
            pl.BlockSpec(memory_space=pl.ANY),
            pl.BlockSpec(memory_space=pl.ANY),
            pl.BlockSpec(memory_space=pl.ANY),
        ],
        out_specs=pl.BlockSpec((T, D_MODEL), lambda: (0, 0)),
        out_shape=jax.ShapeDtypeStruct((T, D_MODEL), jnp.float32),
        scratch_shapes=[
            pltpu.VMEM((NBUF, CF, D_MODEL), jnp.float32),
            pltpu.VMEM((NBUF, CF, D_MODEL), jnp.float32),
            pltpu.VMEM((NBUF, D_MODEL, CF), jnp.float32),
            pltpu.SemaphoreType.DMA((NBUF, 3)),
        ],
        compiler_params=pltpu.CompilerParams(
            vmem_limit_bytes=128 * 1024 * 1024,
        ),
    )(x, wg_pad, W1, W3, W2)
    return out.reshape(inputs.shape)
